# Initial kernel scaffold; baseline (speedup 1.0000x reference)
import jax
import jax.numpy as jnp
from jax.experimental import pallas as pl


def kernel(input_ids, table):
    raise NotImplementedError("write your pallas kernel here")


# speedup vs baseline: 5.2027x; 5.2027x over previous
import functools

import jax
import jax.numpy as jnp
from jax import lax
from jax.experimental import pallas as pl
from jax.experimental.pallas import tpu as pltpu
from jax.experimental.pallas import tpu_sc as plsc

_INFO = plsc.get_sparse_core_info()
_NC = _INFO.num_cores
_NS = _INFO.num_subcores
_L = _INFO.num_lanes
_NW = _NC * _NS

_ROWS, _COLS = 16384, 200
_D = 4
_N_IDS = _ROWS * _COLS
_IDS_PER_W = _N_IDS // _NW
_CHUNK = 4096
_N_CHUNKS = _IDS_PER_W // _CHUNK


def _make_emb():
    mesh = plsc.VectorSubcoreMesh(core_axis_name="c", subcore_axis_name="s")

    @functools.partial(
        pl.kernel,
        mesh=mesh,
        out_type=jax.ShapeDtypeStruct((_N_IDS * _D,), jnp.float32),
        compiler_params=pltpu.CompilerParams(needs_layout_passes=False),
        scratch_types=[
            pltpu.VMEM((8 * _D,), jnp.float32),
            pltpu.VMEM((_CHUNK,), jnp.int32),
            pltpu.VMEM((_CHUNK * _D,), jnp.float32),
        ],
    )
    def emb(tab_hbm, ids_hbm, out_hbm, tab_v, ids_v, out_v):
        wid = lax.axis_index("s") * _NC + lax.axis_index("c")
        base = wid * _IDS_PER_W
        pltpu.sync_copy(tab_hbm, tab_v)
        iota = lax.iota(jnp.int32, _L)

        def chunk_body(c, carry):
            off = base + c * _CHUNK
            pltpu.sync_copy(ids_hbm.at[pl.ds(off, _CHUNK)], ids_v)

            def body(i, carry2):
                ids16 = ids_v[pl.ds(i * _L, _L)]
                a = ids16 * _D
                pos = iota * _D + i * (_L * _D)
                for d in range(_D):
                    g = plsc.load_gather(tab_v, [a + d])
                    plsc.store_scatter(out_v, [pos + d], g)
                return carry2

            lax.fori_loop(0, _CHUNK // _L, body, 0, unroll=4)
            pltpu.sync_copy(out_v, out_hbm.at[pl.ds(off * _D, _CHUNK * _D)])
            return carry

        lax.fori_loop(0, _N_CHUNKS, chunk_body, 0)

    return emb


_emb = _make_emb()


@jax.jit
def kernel(input_ids, table):
    out = _emb(table.reshape(-1), input_ids.reshape(-1))
    return out.reshape(_ROWS, _COLS, _D)


# speedup vs baseline: 31.3101x; 6.0180x over previous
import functools

import jax
import jax.numpy as jnp
from jax import lax
from jax.experimental import pallas as pl
from jax.experimental.pallas import tpu as pltpu
from jax.experimental.pallas import tpu_sc as plsc

_INFO = plsc.get_sparse_core_info()
_NC = _INFO.num_cores
_NS = _INFO.num_subcores
_L = _INFO.num_lanes
_NW = _NC * _NS

_ROWS, _COLS = 16384, 200
_D = 4
_JH = _COLS // 8
_IH = _ROWS // 128
_UNITS_PER_W = (_JH * 32) // _NW


def _make_emb():
    mesh = plsc.VectorSubcoreMesh(core_axis_name="c", subcore_axis_name="s")

    @functools.partial(
        pl.kernel,
        mesh=mesh,
        out_type=jax.ShapeDtypeStruct((_COLS, _IH, _D, 128), jnp.float32),
        compiler_params=pltpu.CompilerParams(needs_layout_passes=False),
        scratch_types=[
            pltpu.VMEM((8, _D), jnp.float32),
            pltpu.VMEM((4, 8, 128), jnp.int32),
            pltpu.VMEM((8, 4, _D, 128), jnp.float32),
        ],
    )
    def emb(tab_hbm, ids_hbm, out_hbm, tab_v, ids_v, out_v):
        wid = lax.axis_index("s") * _NC + lax.axis_index("c")
        pltpu.sync_copy(tab_hbm, tab_v)
        dvecs = [jnp.full((_L,), d, jnp.int32) for d in range(_D)]

        def unit_body(c, carry):
            u = wid * _UNITS_PER_W + c
            jh = u >> 5
            q = u & 31
            pltpu.sync_copy(ids_hbm.at[jh, pl.ds(q * 4, 4)], ids_v)

            def body(t, carry2):
                ti = t >> 6
                jl = (t >> 3) & 7
                s = (t & 7) * _L
                ids16 = ids_v[ti, jl, pl.ds(s, _L)]
                for d in range(_D):
                    g = plsc.load_gather(tab_v, [ids16, dvecs[d]])
                    out_v[jl, ti, d, pl.ds(s, _L)] = g
                return carry2

            lax.fori_loop(0, 256, body, 0, unroll=8)
            pltpu.sync_copy(
                out_v, out_hbm.at[pl.ds(jh * 8, 8), pl.ds(q * 4, 4)]
            )
            return carry

        lax.fori_loop(0, _UNITS_PER_W, unit_body, 0)

    return emb


_emb = _make_emb()


@jax.jit
def kernel(input_ids, table):
    ids4 = input_ids.T.reshape(_JH, 8, _IH, 128).transpose(0, 2, 1, 3)
    out4 = _emb(table, ids4)
    return out4.transpose(1, 3, 0, 2).reshape(_ROWS, _COLS, _D)


# speedup vs baseline: 46.6814x; 1.4909x over previous
import functools

import jax
import jax.numpy as jnp
from jax import lax
from jax.experimental import pallas as pl
from jax.experimental.pallas import tpu as pltpu
from jax.experimental.pallas import tpu_sc as plsc

_INFO = plsc.get_sparse_core_info()
_NC = _INFO.num_cores
_NS = _INFO.num_subcores
_L = _INFO.num_lanes
_NW = _NC * _NS

_ROWS, _COLS = 16384, 200
_D = 4
_JH = _COLS // 8
_IH = _ROWS // 128
_UNITS_PER_W = (_JH * 32) // _NW


def _make_emb():
    mesh = plsc.VectorSubcoreMesh(core_axis_name="c", subcore_axis_name="s")

    @functools.partial(
        pl.kernel,
        mesh=mesh,
        out_type=jax.ShapeDtypeStruct((_COLS, _IH, _D, 128), jnp.float32),
        compiler_params=pltpu.CompilerParams(needs_layout_passes=False),
        scratch_types=[
            pltpu.VMEM((8, _D), jnp.float32),
            pltpu.VMEM((4, 8, 128), jnp.int32),
            pltpu.VMEM((8, 4, _D, 128), jnp.float32),
        ],
    )
    def emb(tab_hbm, ids_hbm, out_hbm, tab_v, ids_v, out_v):
        wid = lax.axis_index("s") * _NC + lax.axis_index("c")
        pltpu.sync_copy(tab_hbm, tab_v)
        dvecs = [jnp.full((_L,), d, jnp.int32) for d in range(_D)]

        def unit_body(c, carry):
            u = wid * _UNITS_PER_W + c
            jh = u >> 5
            q = u & 31
            pltpu.sync_copy(ids_hbm.at[jh, pl.ds(q * 4, 4)], ids_v)

            @plsc.parallel_loop(0, 256, unroll=8)
            def body(t):
                ti = t >> 6
                jl = (t >> 3) & 7
                s = (t & 7) * _L
                ids16 = ids_v[ti, jl, pl.ds(s, _L)]
                for d in range(_D):
                    g = plsc.load_gather(tab_v, [ids16, dvecs[d]])
                    out_v[jl, ti, d, pl.ds(s, _L)] = g
            pltpu.sync_copy(
                out_v, out_hbm.at[pl.ds(jh * 8, 8), pl.ds(q * 4, 4)]
            )
            return carry

        lax.fori_loop(0, _UNITS_PER_W, unit_body, 0)

    return emb


_emb = _make_emb()


@jax.jit
def kernel(input_ids, table):
    ids4 = input_ids.T.reshape(_JH, 8, _IH, 128).transpose(0, 2, 1, 3)
    out4 = _emb(table, ids4)
    return out4.transpose(1, 3, 0, 2).reshape(_ROWS, _COLS, _D)


# speedup vs baseline: 53.7061x; 1.1505x over previous
import functools

import jax
import jax.numpy as jnp
from jax import lax
from jax.experimental import pallas as pl
from jax.experimental.pallas import tpu as pltpu
from jax.experimental.pallas import tpu_sc as plsc

_INFO = plsc.get_sparse_core_info()
_NC = _INFO.num_cores
_NS = _INFO.num_subcores
_L = _INFO.num_lanes
_NW = _NC * _NS

_ROWS, _COLS = 16384, 200
_D = 4
_JH = _COLS // 8
_IH = _ROWS // 128
_UNITS_PER_W = (_JH * 32) // _NW


def _make_emb():
    mesh = plsc.VectorSubcoreMesh(core_axis_name="c", subcore_axis_name="s")

    @functools.partial(
        pl.kernel,
        mesh=mesh,
        out_type=jax.ShapeDtypeStruct((_COLS, _IH, _D, 128), jnp.float32),
        compiler_params=pltpu.CompilerParams(needs_layout_passes=False),
        scratch_types=[
            pltpu.VMEM((8, _D), jnp.float32),
            pltpu.VMEM((2, 4, 8, 128), jnp.int32),
            pltpu.VMEM((2, 8, 4, _D, 128), jnp.float32),
            pltpu.SemaphoreType.DMA((2,)),
            pltpu.SemaphoreType.DMA((2,)),
        ],
    )
    def emb(tab_hbm, ids_hbm, out_hbm, tab_v, ids_v, out_v, isem, osem):
        wid = lax.axis_index("s") * _NC + lax.axis_index("c")
        pltpu.sync_copy(tab_hbm, tab_v)
        dvecs = [jnp.full((_L,), d, jnp.int32) for d in range(_D)]

        def ids_dma(c, buf):
            u = wid * _UNITS_PER_W + c
            jh = u >> 5
            q = u & 31
            return pltpu.make_async_copy(
                ids_hbm.at[jh, pl.ds(q * 4, 4)], ids_v.at[buf], isem.at[buf]
            )

        def out_dma(c, buf):
            u = wid * _UNITS_PER_W + c
            jh = u >> 5
            q = u & 31
            return pltpu.make_async_copy(
                out_v.at[buf],
                out_hbm.at[pl.ds(jh * 8, 8), pl.ds(q * 4, 4)],
                osem.at[buf],
            )

        ids_dma(0, 0).start()

        def unit_body(c, carry):
            cur = c & 1

            @pl.when(c + 1 < _UNITS_PER_W)
            def _():
                ids_dma(c + 1, 1 - cur).start()

            ids_dma(c, cur).wait()

            @pl.when(c >= 2)
            def _():
                out_dma(c - 2, cur).wait()

            @plsc.parallel_loop(0, 256, unroll=8)
            def body(t):
                ti = t >> 6
                jl = (t >> 3) & 7
                s = (t & 7) * _L
                ids16 = ids_v[cur, ti, jl, pl.ds(s, _L)]
                for d in range(_D):
                    g = plsc.load_gather(tab_v, [ids16, dvecs[d]])
                    out_v[cur, jl, ti, d, pl.ds(s, _L)] = g

            out_dma(c, cur).start()
            return carry

        lax.fori_loop(0, _UNITS_PER_W, unit_body, 0)
        out_dma(_UNITS_PER_W - 2, (_UNITS_PER_W - 2) & 1).wait()
        out_dma(_UNITS_PER_W - 1, (_UNITS_PER_W - 1) & 1).wait()

    return emb


_emb = _make_emb()


@jax.jit
def kernel(input_ids, table):
    ids4 = input_ids.T.reshape(_JH, 8, _IH, 128).transpose(0, 2, 1, 3)
    out4 = _emb(table, ids4)
    return out4.transpose(1, 3, 0, 2).reshape(_ROWS, _COLS, _D)
